# baseline (speedup 1.0000x reference)
import math

import jax
import jax.numpy as jnp
from jax.experimental import pallas as pl

BS, M, P = 32, 256, 20
PP = 24
N = BS * M
NB = 256
NBP = NB * PP


def _encoder_kernel(pv_ref, orient_ref, vm_ref, center_ref, sl_ref, idx_ref,
                    w1a_ref, w1cs_ref, b1_ref, w2_ref, b2_ref,
                    w3a_ref, w3b_ref, b3_ref, w4_ref, b4_ref,
                    fr_ref, wf1c_ref, wf1s_ref, wf1l_ref, bf1_ref,
                    lng_ref, lnb_ref, wf2_ref, bf2_ref, t9_ref,
                    out1_ref, out2_ref):
    f32 = jnp.float32

    c4 = center_ref[...]
    c4b = jnp.broadcast_to(c4.reshape(NB, 1, 4), (NB, PP, 4)).reshape(NBP, 4)
    x4 = pv_ref[...] - c4b
    o = orient_ref[...]

    h1 = jnp.dot(x4, w1a_ref[...], preferred_element_type=f32)
    h1 = h1 + jnp.cos(o) * w1cs_ref[0:1, :] + jnp.sin(o) * w1cs_ref[1:2, :]
    h1 = jax.nn.relu(h1 + b1_ref[...])

    bf16 = jnp.bfloat16
    h = jnp.dot(h1.astype(bf16), w2_ref[...], preferred_element_type=f32)
    h = h + b2_ref[...]
    vm = vm_ref[...]
    hm = h * vm

    row = jax.lax.broadcasted_iota(jnp.int32, (NBP, 1), 0)
    pb = jnp.where(row % PP >= P, f32(-1e30), f32(0.0))

    pooled = jnp.max((hm + pb).reshape(NB, PP, 256), axis=1)

    q = jnp.dot(pooled.astype(bf16), w3b_ref[...], preferred_element_type=f32)
    q = q + b3_ref[...]
    qb = jnp.broadcast_to(q.reshape(NB, 1, 256), (NB, PP, 256)).reshape(NBP, 256)
    t = jnp.dot(hm.astype(bf16), w3a_ref[...], preferred_element_type=f32) + qb
    t = jax.nn.relu(t)

    z = jnp.dot(t.astype(bf16), w4_ref[...], preferred_element_type=f32)
    z = z + b4_ref[...]
    zb = z * vm + pb
    x_poly = jnp.max(zb.reshape(NB, PP, 128), axis=1)

    sl = sl_ref[...]
    xf = sl * fr_ref[...]
    fh = (jnp.dot(jnp.cos(xf), wf1c_ref[...], preferred_element_type=f32)
          + jnp.dot(jnp.sin(xf), wf1s_ref[...], preferred_element_type=f32)
          + sl * wf1l_ref[...] + bf1_ref[...])
    mu = jnp.mean(fh, axis=-1, keepdims=True)
    var = jnp.mean((fh - mu) ** 2, axis=-1, keepdims=True)
    fh = (fh - mu) / jnp.sqrt(var + 1e-5) * lng_ref[...] + lnb_ref[...]
    fh = jax.nn.relu(fh)
    fh = jnp.dot(fh, wf2_ref[...], preferred_element_type=f32) + bf2_ref[...]
    hasl = (idx_ref[:, 3:4] > 0).astype(f32)
    xs = fh * hasl

    ty = idx_ref[:, 0:1]
    rt = idx_ref[:, 1:2]
    tl = idx_ref[:, 2:3]
    t1 = ty == 1
    tl_inv = jnp.where((tl == 2) & t1, 0, tl)
    tl_inv = jnp.where((tl == 1) & t1, 0, tl_inv)
    tl_inv = jnp.where((tl == 0) & t1, 2, tl_inv)
    io9 = jax.lax.broadcasted_iota(jnp.int32, (NB, 9), 1)
    hf = ((io9 == ty) | (io9 == rt + 3) | (io9 == tl + 5)).astype(f32)
    hi = ((io9 == ty) | (io9 == rt + 3) | (io9 == tl_inv + 5)).astype(f32)
    ef = jnp.dot(hf, t9_ref[...], preferred_element_type=f32)
    ei = jnp.dot(hi, t9_ref[...], preferred_element_type=f32)

    base = x_poly + xs
    out1_ref[...] = base + ef
    out2_ref[...] = base + ei


def kernel(polygon_center, polygon_speed_limit, point_position, point_vector,
           point_orientation, polygon_type, polygon_on_route, polygon_tl_status,
           polygon_has_speed_limit, valid_mask, W1, b1, g1, be1, m1, v1, W2, b2,
           W3, b3, g3, be3, m3, v3, W4, b4, freqs, Wf1, bf1, lng, lnb, Wf2, bf2,
           type_table, route_table, tl_table):
    f32 = jnp.float32

    pv = jnp.concatenate([point_position[:, :, 0], point_vector[:, :, 0]],
                         axis=-1).reshape(N, P, 4)
    pv = jnp.pad(pv, ((0, 0), (0, PP - P), (0, 0))).reshape(N * PP, 4)
    orient = jnp.pad(point_orientation[:, :, 0].reshape(N, P),
                     ((0, 0), (0, PP - P))).reshape(N * PP, 1)
    vm = jnp.pad(valid_mask.astype(f32).reshape(N, P),
                 ((0, 0), (0, PP - P))).reshape(N * PP, 1)
    center = jnp.pad(polygon_center.reshape(N, 3)[:, :2], ((0, 0), (0, 2)))
    sl = polygon_speed_limit.reshape(N, 1)
    idx = jnp.stack([polygon_type.reshape(N), polygon_on_route.reshape(N),
                     polygon_tl_status.reshape(N),
                     polygon_has_speed_limit.reshape(N).astype(jnp.int32)],
                    axis=-1).astype(jnp.int32)

    s1 = g1 / jnp.sqrt(v1 + 1e-5)
    W1f = W1 * s1
    b1f = b1 * s1 + (be1 - m1 * s1)
    s3 = g3 / jnp.sqrt(v3 + 1e-5)
    W3f = W3 * s3
    b3f = b3 * s3 + (be3 - m3 * s3)

    bf16 = jnp.bfloat16
    w1a = W1f[0:4]
    w1cs = W1f[4:6]
    W2c = W2.astype(bf16)
    w3a = W3f[0:256].astype(bf16)
    w3b = W3f[256:512].astype(bf16)
    W4c = W4.astype(bf16)
    fr = (freqs * (2.0 * math.pi)).reshape(1, 64)
    wf1c = Wf1[0:64]
    wf1s = Wf1[64:128]
    wf1l = Wf1[128:129]
    t9 = jnp.concatenate([type_table, route_table, tl_table], axis=0)

    row2 = lambda a: a.reshape(1, -1)

    grid = N // NB
    full = lambda shape: pl.BlockSpec(shape, lambda i: (0, 0))
    out1, out2 = pl.pallas_call(
        _encoder_kernel,
        grid=(grid,),
        in_specs=[
            pl.BlockSpec((NBP, 4), lambda i: (i, 0)),
            pl.BlockSpec((NBP, 1), lambda i: (i, 0)),
            pl.BlockSpec((NBP, 1), lambda i: (i, 0)),
            pl.BlockSpec((NB, 4), lambda i: (i, 0)),
            pl.BlockSpec((NB, 1), lambda i: (i, 0)),
            pl.BlockSpec((NB, 4), lambda i: (i, 0)),
            full((4, 128)), full((2, 128)), full((1, 128)),
            full((128, 256)), full((1, 256)),
            full((256, 256)), full((256, 256)), full((1, 256)),
            full((256, 128)), full((1, 128)),
            full((1, 64)), full((64, 128)), full((64, 128)),
            full((1, 128)), full((1, 128)),
            full((1, 128)), full((1, 128)),
            full((128, 128)), full((1, 128)),
            full((9, 128)),
        ],
        out_specs=[
            pl.BlockSpec((NB, 128), lambda i: (i, 0)),
            pl.BlockSpec((NB, 128), lambda i: (i, 0)),
        ],
        out_shape=[
            jax.ShapeDtypeStruct((N, 128), f32),
            jax.ShapeDtypeStruct((N, 128), f32),
        ],
    )(pv, orient, vm, center, sl, idx,
      w1a, w1cs, row2(b1f), W2c, row2(b2), w3a, w3b, row2(b3f), W4c, row2(b4),
      fr, wf1c, wf1s, wf1l, row2(bf1), row2(lng), row2(lnb), Wf2, row2(bf2), t9)

    return out1.reshape(BS, M, 128), out2.reshape(BS, M, 128)


# speedup vs baseline: 1.9279x; 1.9279x over previous
import math

import jax
import jax.numpy as jnp
from jax.experimental import pallas as pl

BS, M, P = 32, 256, 20
PP = 24
N = BS * M
NB = 256
NBP = NB * PP


def _encoder_kernel(pv_ref, vm_ref, center_ref, sl_ref, idx_ref,
                    w1a_ref, b1_ref, w2_ref, b2_ref,
                    w3a_ref, w3b_ref, b3_ref, w4_ref, b4_ref,
                    fr_ref, wf1c_ref, wf1s_ref, wf1l_ref, bf1_ref,
                    lng_ref, lnb_ref, wf2_ref, bf2_ref, t9_ref,
                    out1_ref, out2_ref):
    f32 = jnp.float32

    c6 = center_ref[...]
    c6b = jnp.broadcast_to(c6.reshape(NB, 1, 6), (NB, PP, 6)).reshape(NBP, 6)
    feat = pv_ref[...] - c6b

    h1 = jnp.dot(feat, w1a_ref[...], preferred_element_type=f32)
    h1 = jax.nn.relu(h1 + b1_ref[...])

    bf16 = jnp.bfloat16
    h = jnp.dot(h1.astype(bf16), w2_ref[...], preferred_element_type=f32)
    h = h + b2_ref[...]
    vm = vm_ref[...]
    hm = h * vm

    row = jax.lax.broadcasted_iota(jnp.int32, (NBP, 1), 0)
    pb = jnp.where(row % PP >= P, f32(-1e30), f32(0.0))

    pooled = jnp.max((hm + pb).reshape(NB, PP, 256), axis=1)

    q = jnp.dot(pooled.astype(bf16), w3b_ref[...], preferred_element_type=f32)
    q = q + b3_ref[...]
    qb = jnp.broadcast_to(q.reshape(NB, 1, 256), (NB, PP, 256)).reshape(NBP, 256)
    t = jnp.dot(hm.astype(bf16), w3a_ref[...], preferred_element_type=f32) + qb
    t = jax.nn.relu(t)

    z = jnp.dot(t.astype(bf16), w4_ref[...], preferred_element_type=f32)
    z = z + b4_ref[...]
    zb = z * vm + pb
    x_poly = jnp.max(zb.reshape(NB, PP, 128), axis=1)

    sl = sl_ref[...]
    xf = sl * fr_ref[...]
    fh = (jnp.dot(jnp.cos(xf), wf1c_ref[...], preferred_element_type=f32)
          + jnp.dot(jnp.sin(xf), wf1s_ref[...], preferred_element_type=f32)
          + sl * wf1l_ref[...] + bf1_ref[...])
    mu = jnp.mean(fh, axis=-1, keepdims=True)
    var = jnp.mean((fh - mu) ** 2, axis=-1, keepdims=True)
    fh = (fh - mu) / jnp.sqrt(var + 1e-5) * lng_ref[...] + lnb_ref[...]
    fh = jax.nn.relu(fh)
    fh = jnp.dot(fh, wf2_ref[...], preferred_element_type=f32) + bf2_ref[...]
    hasl = (idx_ref[:, 3:4] > 0).astype(f32)
    xs = fh * hasl

    ty = idx_ref[:, 0:1]
    rt = idx_ref[:, 1:2]
    tl = idx_ref[:, 2:3]
    t1 = ty == 1
    tl_inv = jnp.where((tl == 2) & t1, 0, tl)
    tl_inv = jnp.where((tl == 1) & t1, 0, tl_inv)
    tl_inv = jnp.where((tl == 0) & t1, 2, tl_inv)
    io9 = jax.lax.broadcasted_iota(jnp.int32, (NB, 9), 1)
    hf = ((io9 == ty) | (io9 == rt + 3) | (io9 == tl + 5)).astype(f32)
    hi = ((io9 == ty) | (io9 == rt + 3) | (io9 == tl_inv + 5)).astype(f32)
    ef = jnp.dot(hf, t9_ref[...], preferred_element_type=f32)
    ei = jnp.dot(hi, t9_ref[...], preferred_element_type=f32)

    base = x_poly + xs
    out1_ref[...] = base + ef
    out2_ref[...] = base + ei


def kernel(polygon_center, polygon_speed_limit, point_position, point_vector,
           point_orientation, polygon_type, polygon_on_route, polygon_tl_status,
           polygon_has_speed_limit, valid_mask, W1, b1, g1, be1, m1, v1, W2, b2,
           W3, b3, g3, be3, m3, v3, W4, b4, freqs, Wf1, bf1, lng, lnb, Wf2, bf2,
           type_table, route_table, tl_table):
    f32 = jnp.float32

    o = point_orientation[:, :, 0]
    pv = jnp.concatenate([point_position[:, :, 0], point_vector[:, :, 0],
                          jnp.cos(o)[..., None], jnp.sin(o)[..., None]],
                         axis=-1).reshape(N, P, 6)
    pv = jnp.pad(pv, ((0, 0), (0, PP - P), (0, 0))).reshape(N * PP, 6)
    vm = jnp.pad(valid_mask.astype(f32).reshape(N, P),
                 ((0, 0), (0, PP - P))).reshape(N * PP, 1)
    center = jnp.pad(polygon_center.reshape(N, 3)[:, :2], ((0, 0), (0, 4)))
    sl = polygon_speed_limit.reshape(N, 1)
    idx = jnp.stack([polygon_type.reshape(N), polygon_on_route.reshape(N),
                     polygon_tl_status.reshape(N),
                     polygon_has_speed_limit.reshape(N).astype(jnp.int32)],
                    axis=-1).astype(jnp.int32)

    s1 = g1 / jnp.sqrt(v1 + 1e-5)
    W1f = W1 * s1
    b1f = b1 * s1 + (be1 - m1 * s1)
    s3 = g3 / jnp.sqrt(v3 + 1e-5)
    W3f = W3 * s3
    b3f = b3 * s3 + (be3 - m3 * s3)

    bf16 = jnp.bfloat16
    w1a = W1f[0:6]
    W2c = W2.astype(bf16)
    w3a = W3f[0:256].astype(bf16)
    w3b = W3f[256:512].astype(bf16)
    W4c = W4.astype(bf16)
    fr = (freqs * (2.0 * math.pi)).reshape(1, 64)
    wf1c = Wf1[0:64]
    wf1s = Wf1[64:128]
    wf1l = Wf1[128:129]
    t9 = jnp.concatenate([type_table, route_table, tl_table], axis=0)

    row2 = lambda a: a.reshape(1, -1)

    grid = N // NB
    full = lambda shape: pl.BlockSpec(shape, lambda i: (0, 0))
    out1, out2 = pl.pallas_call(
        _encoder_kernel,
        grid=(grid,),
        in_specs=[
            pl.BlockSpec((NBP, 6), lambda i: (i, 0)),
            pl.BlockSpec((NBP, 1), lambda i: (i, 0)),
            pl.BlockSpec((NB, 6), lambda i: (i, 0)),
            pl.BlockSpec((NB, 1), lambda i: (i, 0)),
            pl.BlockSpec((NB, 4), lambda i: (i, 0)),
            full((6, 128)), full((1, 128)),
            full((128, 256)), full((1, 256)),
            full((256, 256)), full((256, 256)), full((1, 256)),
            full((256, 128)), full((1, 128)),
            full((1, 64)), full((64, 128)), full((64, 128)),
            full((1, 128)), full((1, 128)),
            full((1, 128)), full((1, 128)),
            full((128, 128)), full((1, 128)),
            full((9, 128)),
        ],
        out_specs=[
            pl.BlockSpec((NB, 128), lambda i: (i, 0)),
            pl.BlockSpec((NB, 128), lambda i: (i, 0)),
        ],
        out_shape=[
            jax.ShapeDtypeStruct((N, 128), f32),
            jax.ShapeDtypeStruct((N, 128), f32),
        ],
    )(pv, vm, center, sl, idx,
      w1a, row2(b1f), W2c, row2(b2), w3a, w3b, row2(b3f), W4c, row2(b4),
      fr, wf1c, wf1s, wf1l, row2(bf1), row2(lng), row2(lnb), Wf2, row2(bf2), t9)

    return out1.reshape(BS, M, 128), out2.reshape(BS, M, 128)
